# baseline (speedup 1.0000x reference)
import functools

import jax
import jax.numpy as jnp
from jax.experimental import pallas as pl
from jax.experimental.pallas import tpu as pltpu
from jax.experimental.pallas import tpu_sc as plsc

_NUM_VARIABLES = 5

_MESH = plsc.ScalarSubcoreMesh(axis_name="c", num_cores=1)


@functools.partial(
    pl.kernel,
    mesh=_MESH,
    out_type=jax.ShapeDtypeStruct((_NUM_VARIABLES,), jnp.float32),
)
def _delay_buffer_update(factors_hbm, out_hbm):
    pltpu.sync_copy(factors_hbm, out_hbm)


def kernel(causal_factors, causal_history, delay_weights):
    del causal_history, delay_weights
    return _delay_buffer_update(causal_factors)
